# baseline (speedup 1.0000x reference)
import functools

import jax
import jax.numpy as jnp
from jax import lax
from jax.experimental import pallas as pl
from jax.experimental.pallas import tpu as pltpu
from jax.experimental.pallas import tpu_sc as plsc

_COMMITMENT_COST = 0.25


def _dist_argmin_body(z_ref, cb_ref, cb2_ref, idx_ref, loss_ref, d_scr,
                      acc_ref):
    i = pl.program_id(0)
    nb = pl.num_programs(0)
    zb = z_ref[0]
    cb = cb_ref[...]
    cb2 = cb2_ref[...]
    zbt = zb.T
    mm2 = lax.dot_general(cb2, zbt, (((1,), (0,)), ((), ())),
                          preferred_element_type=jnp.float32)
    cn = jnp.sum(cb * cb, axis=1, keepdims=True)
    zn = jnp.sum(zbt * zbt, axis=0, keepdims=True)
    d_scr[...] = (zn + cn) + mm2
    k, p = d_scr.shape

    minv0 = d_scr[0:8, :]
    mini0 = jnp.zeros((8, p), jnp.int32)

    def step(r, carry):
        minv, mini = carry
        dr = d_scr[pl.ds(r * 8, 8), :]
        better = dr < minv
        minv = jnp.where(better, dr, minv)
        mini = jnp.where(better, r, mini)
        return minv, mini

    minv, mini = lax.fori_loop(1, k // 8, step, (minv0, mini0), unroll=1)
    slane = lax.broadcasted_iota(jnp.int32, (8, p), 0)
    idx8 = mini * 8 + slane
    m = jnp.min(minv, axis=0, keepdims=True)
    idx = jnp.min(jnp.where(minv == m, idx8, jnp.int32(k)), axis=0)
    idx_ref[0, 0, :] = idx

    @pl.when(i == 0)
    def _init():
        acc_ref[0] = jnp.float32(0.0)

    acc_ref[0] += jnp.sum(m)

    @pl.when(i == nb - 1)
    def _fin():
        mse = acc_ref[0] / (nb * p * zb.shape[1])
        loss_ref[0] = mse + _COMMITMENT_COST * mse


def _dist_argmin(zf, codebook):
    b, p, dim = zf.shape
    k = codebook.shape[0]
    return pl.pallas_call(
        _dist_argmin_body,
        grid=(b,),
        in_specs=[
            pl.BlockSpec((1, p, dim), lambda i: (i, 0, 0)),
            pl.BlockSpec((k, dim), lambda i: (0, 0)),
            pl.BlockSpec((k, dim), lambda i: (0, 0)),
        ],
        out_specs=[
            pl.BlockSpec((1, 1, p), lambda i: (i, 0, 0)),
            pl.BlockSpec(memory_space=pltpu.SMEM),
        ],
        out_shape=[
            jax.ShapeDtypeStruct((b, 1, p), jnp.int32),
            jax.ShapeDtypeStruct((1,), jnp.float32),
        ],
        scratch_shapes=[
            pltpu.VMEM((k, p), jnp.float32),
            pltpu.SMEM((1,), jnp.float32),
        ],
    )(zf, codebook, jnp.float32(-2.0) * codebook)


def _make_sc_gather(n, dim, nc, nw, chunks, chunk):
    b_per_w = chunks * chunk
    mesh = plsc.VectorSubcoreMesh(core_axis_name="c", subcore_axis_name="s")

    @functools.partial(
        pl.kernel, mesh=mesh,
        compiler_params=pltpu.CompilerParams(use_tc_tiling_on_sc=False),
        out_type=jax.ShapeDtypeStruct((n, dim), jnp.float32),
        scratch_types=[
            pltpu.VMEM((chunks, chunk), jnp.int32),
            pltpu.VMEM((b_per_w, dim), jnp.float32),
            pltpu.SemaphoreType.DMA,
        ],
    )
    def gather_kernel(table_hbm, idx_hbm, out_hbm, idx_v, rows_v, sem):
        wid = lax.axis_index("s") * nc + lax.axis_index("c")
        pltpu.sync_copy(idx_hbm.at[wid], idx_v)
        copies = []
        for j in range(chunks):
            copies.append(pltpu.async_copy(
                table_hbm.at[idx_v.at[j]],
                rows_v.at[pl.ds(j * chunk, chunk)], sem))
        for c in copies:
            c.wait()
        pltpu.sync_copy(rows_v, out_hbm.at[pl.ds(wid * b_per_w, b_per_w)])

    return gather_kernel


def kernel(z, codebook):
    b, dim, h, w = z.shape
    p = h * w
    n = b * p
    zf = z.transpose(0, 2, 3, 1).reshape(b // 2, 2 * p, dim)
    idx3, loss1 = _dist_argmin(zf, codebook)
    idx_flat = idx3.reshape(n)
    loss = loss1[0]

    info = plsc.get_sparse_core_info()
    nc, ns = info.num_cores, info.num_subcores
    nw = nc * ns
    chunk = 128
    chunks = n // (nw * chunk)
    gather_fn = _make_sc_gather(n, dim, nc, nw, chunks, chunk)
    zq_flat = gather_fn(codebook, idx_flat.reshape(nw, chunks, chunk))

    z_q = zq_flat.reshape(b, h, w, dim).transpose(0, 3, 1, 2)
    return z_q, loss, idx_flat


# speedup vs baseline: 1.0872x; 1.0872x over previous
import functools

import jax
import jax.numpy as jnp
from jax import lax
from jax.experimental import pallas as pl
from jax.experimental.pallas import tpu as pltpu
from jax.experimental.pallas import tpu_sc as plsc

_COMMITMENT_COST = 0.25


def _dist_argmin_body(z_ref, cb_ref, cb2_ref, idx_ref, loss_ref, d_scr,
                      acc_ref):
    i = pl.program_id(0)
    nb = pl.num_programs(0)
    zb = z_ref[0]
    cb = cb_ref[...]
    cb2 = cb2_ref[...]
    zbt = zb.T
    mm2 = lax.dot_general(cb2, zbt, (((1,), (0,)), ((), ())),
                          preferred_element_type=jnp.float32)
    cn = jnp.sum(cb * cb, axis=1, keepdims=True)
    zn = jnp.sum(zbt * zbt, axis=0, keepdims=True)
    d_scr[...] = (zn + cn) + mm2
    k, p = d_scr.shape

    minv0 = d_scr[0:8, :]
    mini0 = jnp.zeros((8, p), jnp.int32)

    def step(r, carry):
        minv, mini = carry
        dr = d_scr[pl.ds(r * 8, 8), :]
        better = dr < minv
        minv = jnp.where(better, dr, minv)
        mini = jnp.where(better, r, mini)
        return minv, mini

    minv, mini = lax.fori_loop(1, k // 8, step, (minv0, mini0), unroll=8)
    slane = lax.broadcasted_iota(jnp.int32, (8, p), 0)
    idx8 = mini * 8 + slane
    m = jnp.min(minv, axis=0, keepdims=True)
    idx = jnp.min(jnp.where(minv == m, idx8, jnp.int32(k)), axis=0)
    idx_ref[0, 0, :] = idx

    @pl.when(i == 0)
    def _init():
        acc_ref[0] = jnp.float32(0.0)

    acc_ref[0] += jnp.sum(m)

    @pl.when(i == nb - 1)
    def _fin():
        mse = acc_ref[0] / (nb * p * zb.shape[1])
        loss_ref[0] = mse + _COMMITMENT_COST * mse


def _dist_argmin(zf, codebook):
    b, p, dim = zf.shape
    k = codebook.shape[0]
    return pl.pallas_call(
        _dist_argmin_body,
        grid=(b,),
        in_specs=[
            pl.BlockSpec((1, p, dim), lambda i: (i, 0, 0)),
            pl.BlockSpec((k, dim), lambda i: (0, 0)),
            pl.BlockSpec((k, dim), lambda i: (0, 0)),
        ],
        out_specs=[
            pl.BlockSpec((1, 1, p), lambda i: (i, 0, 0)),
            pl.BlockSpec(memory_space=pltpu.SMEM),
        ],
        out_shape=[
            jax.ShapeDtypeStruct((b, 1, p), jnp.int32),
            jax.ShapeDtypeStruct((1,), jnp.float32),
        ],
        scratch_shapes=[
            pltpu.VMEM((k, p), jnp.float32),
            pltpu.SMEM((1,), jnp.float32),
        ],
    )(zf, codebook, jnp.float32(-2.0) * codebook)


def _make_sc_gather(n, dim, nc, nw, chunks, chunk):
    b_per_w = chunks * chunk
    mesh = plsc.VectorSubcoreMesh(core_axis_name="c", subcore_axis_name="s")

    @functools.partial(
        pl.kernel, mesh=mesh,
        compiler_params=pltpu.CompilerParams(use_tc_tiling_on_sc=False),
        out_type=jax.ShapeDtypeStruct((n, dim), jnp.float32),
        scratch_types=[
            pltpu.VMEM((chunks, chunk), jnp.int32),
            pltpu.VMEM((b_per_w, dim), jnp.float32),
            pltpu.SemaphoreType.DMA,
        ],
    )
    def gather_kernel(table_hbm, idx_hbm, out_hbm, idx_v, rows_v, sem):
        wid = lax.axis_index("s") * nc + lax.axis_index("c")
        pltpu.sync_copy(idx_hbm.at[wid], idx_v)
        copies = []
        for j in range(chunks):
            copies.append(pltpu.async_copy(
                table_hbm.at[idx_v.at[j]],
                rows_v.at[pl.ds(j * chunk, chunk)], sem))
        for c in copies:
            c.wait()
        pltpu.sync_copy(rows_v, out_hbm.at[pl.ds(wid * b_per_w, b_per_w)])

    return gather_kernel


def kernel(z, codebook):
    b, dim, h, w = z.shape
    p = h * w
    n = b * p
    zf = z.transpose(0, 2, 3, 1).reshape(b // 2, 2 * p, dim)
    idx3, loss1 = _dist_argmin(zf, codebook)
    idx_flat = idx3.reshape(n)
    loss = loss1[0]

    info = plsc.get_sparse_core_info()
    nc, ns = info.num_cores, info.num_subcores
    nw = nc * ns
    chunk = 128
    chunks = n // (nw * chunk)
    gather_fn = _make_sc_gather(n, dim, nc, nw, chunks, chunk)
    zq_flat = gather_fn(codebook, idx_flat.reshape(nw, chunks, chunk))

    z_q = zq_flat.reshape(b, h, w, dim).transpose(0, 3, 1, 2)
    return z_q, loss, idx_flat
